# Initial kernel scaffold; baseline (speedup 1.0000x reference)
import jax
import jax.numpy as jnp
from jax.experimental import pallas as pl


def kernel(x, W_shared, b_shared, W_gate, b_gate, W1, b1, W2, b2):
    raise NotImplementedError("write your pallas kernel here")


# speedup vs baseline: 1.4109x; 1.4109x over previous
import functools

import jax
import jax.numpy as jnp
from jax.experimental import pallas as pl
from jax.experimental.pallas import tpu as pltpu


def _moe_kernel(x_ref, ws_ref, bs_ref, wg_ref, bg_ref, w1_ref, b1_ref,
                w2_ref, b2_ref, res_ref, f_ref, p_ref,
                acc_ref, w_ref, fcnt_ref, selp_ref,
                *, n_e, n_t, tb, n_tok):
    e = pl.program_id(0)
    t = pl.program_id(1)
    x_blk = x_ref[...]
    tsl = pl.ds(t * tb, tb)

    @pl.when(e == 0)
    def _gate():
        ncand = wg_ref.shape[1]
        s = jax.nn.sigmoid(
            jnp.dot(x_blk, wg_ref[...], preferred_element_type=jnp.float32)
            + bg_ref[...])
        iota = jax.lax.broadcasted_iota(jnp.int32, s.shape, 1)
        m1 = jnp.max(s, axis=1, keepdims=True)
        i1 = jnp.min(jnp.where(s == m1, iota, ncand), axis=1, keepdims=True)
        sm = jnp.where(iota == i1, -jnp.inf, s)
        m2 = jnp.max(sm, axis=1, keepdims=True)
        i2 = jnp.min(jnp.where(sm == m2, iota, ncand), axis=1, keepdims=True)
        gsum = m1 + m2
        sel1 = iota == i1
        sel2 = iota == i2
        w = jnp.where(sel1, m1 / gsum, 0.0) + jnp.where(sel2, m2 / gsum, 0.0)
        w_ref[tsl, :] = w
        sn = s / jnp.sum(s, axis=1, keepdims=True)
        selmask = jnp.logical_or(sel1, sel2)
        cnt_blk = jnp.sum(selmask.astype(jnp.float32), axis=0, keepdims=True)
        selp_blk = jnp.sum(jnp.where(selmask, sn, 0.0), axis=0, keepdims=True)

        @pl.when(t == 0)
        def _init():
            fcnt_ref[...] = jnp.zeros_like(fcnt_ref)
            selp_ref[...] = jnp.zeros_like(selp_ref)

        fcnt_ref[...] += cnt_blk
        selp_ref[...] += selp_blk

        @pl.when(t == n_t - 1)
        def _stats_out():
            fcnt = fcnt_ref[...]
            selp = selp_ref[...]
            f_ref[...] = 2.0 * n_tok - fcnt
            p_ref[...] = jnp.sum(selp) - selp

        sh = jnp.dot(x_blk, ws_ref[...], preferred_element_type=jnp.float32)
        acc_ref[tsl, :] = x_blk + sh + bs_ref[...]

    iota_e = jax.lax.broadcasted_iota(jnp.int32, (tb, w_ref.shape[1]), 1)
    w_col = jnp.sum(jnp.where(iota_e == e, w_ref[tsl, :], 0.0), axis=1,
                    keepdims=True)
    h = jax.nn.gelu(
        jnp.dot(x_blk, w1_ref[0], preferred_element_type=jnp.float32)
        + b1_ref[0])
    eo = jnp.dot(h, w2_ref[0], preferred_element_type=jnp.float32) + b2_ref[0]
    acc_ref[tsl, :] += w_col * eo

    @pl.when(e == n_e - 1)
    def _out():
        res_ref[...] = acc_ref[tsl, :]


def kernel(x, W_shared, b_shared, W_gate, b_gate, W1, b1, W2, b2):
    B, T, C = x.shape
    E, _, FF = W1.shape
    TB = 256
    NT = (B * T) // TB
    x2 = x.reshape(B * T, C)

    grid = (E, NT)
    kfn = functools.partial(_moe_kernel, n_e=E, n_t=NT, tb=TB, n_tok=B * T)
    res, f, p = pl.pallas_call(
        kfn,
        grid=grid,
        in_specs=[
            pl.BlockSpec((TB, C), lambda e, t: (t, 0)),
            pl.BlockSpec((C, C), lambda e, t: (0, 0)),
            pl.BlockSpec((1, C), lambda e, t: (0, 0)),
            pl.BlockSpec((C, E), lambda e, t: (0, 0)),
            pl.BlockSpec((1, E), lambda e, t: (0, 0)),
            pl.BlockSpec((1, C, FF), lambda e, t: (e, 0, 0)),
            pl.BlockSpec((1, 1, FF), lambda e, t: (e, 0, 0)),
            pl.BlockSpec((1, FF, C), lambda e, t: (e, 0, 0)),
            pl.BlockSpec((1, 1, C), lambda e, t: (e, 0, 0)),
        ],
        out_specs=[
            pl.BlockSpec((TB, C), lambda e, t: (t, 0)),
            pl.BlockSpec((1, E), lambda e, t: (0, 0)),
            pl.BlockSpec((1, E), lambda e, t: (0, 0)),
        ],
        out_shape=[
            jax.ShapeDtypeStruct((B * T, C), jnp.float32),
            jax.ShapeDtypeStruct((1, E), jnp.float32),
            jax.ShapeDtypeStruct((1, E), jnp.float32),
        ],
        scratch_shapes=[
            pltpu.VMEM((B * T, C), jnp.float32),
            pltpu.VMEM((B * T, E), jnp.float32),
            pltpu.VMEM((1, E), jnp.float32),
            pltpu.VMEM((1, E), jnp.float32),
        ],
    )(x2, W_shared, b_shared.reshape(1, C), W_gate, b_gate.reshape(1, E),
      W1, b1.reshape(E, 1, FF), W2, b2.reshape(E, 1, C))
    return res.reshape(B, T, C), f, p
